# Initial kernel scaffold; baseline (speedup 1.0000x reference)
import jax
import jax.numpy as jnp
from jax.experimental import pallas as pl


def kernel(x, batch, query, W_k, b_k, W_v, b_v):
    raise NotImplementedError("write your pallas kernel here")


# speedup vs baseline: 38.8937x; 38.8937x over previous
import functools
import jax
import jax.numpy as jnp
from jax import lax
from jax.experimental import pallas as pl
from jax.experimental.pallas import tpu as pltpu

DIM = 128
H = 4
HD = 32
BSZ = 1024
N_ROWS = 100000
BLK = 800


def _tc_body(batch_ref, x_ref, wvt_ref, a_ref, c_ref, bv_ref,
             out_ref, accp, accs):
    i = pl.program_id(0)

    @pl.when(i == 0)
    def _init():
        accp[...] = jnp.zeros_like(accp)
        accs[...] = jnp.zeros_like(accs)

    x = x_ref[...]
    v = jnp.dot(x, wvt_ref[...],
                preferred_element_type=jnp.float32) + bv_ref[...]
    attn = jnp.dot(x, a_ref[...],
                   preferred_element_type=jnp.float32) + c_ref[...]
    e = jnp.exp(attn)

    hrow = lax.broadcasted_iota(jnp.int32, (8, DIM), 0)
    hcol = lax.broadcasted_iota(jnp.int32, (8, DIM), 1) // HD
    bmat = (hrow == hcol).astype(jnp.float32)
    eb = jnp.dot(e, bmat, preferred_element_type=jnp.float32)
    ev = eb * v

    brow = batch_ref[0]
    seg_iota = lax.broadcasted_iota(jnp.int32, (BSZ, BLK), 0).astype(jnp.float32)
    oht = (jnp.broadcast_to(brow, (BSZ, BLK)) == seg_iota).astype(jnp.float32)

    accp[...] += jnp.dot(oht, ev, preferred_element_type=jnp.float32)
    accs[...] += jnp.dot(oht, e, preferred_element_type=jnp.float32)

    @pl.when(i == pl.num_programs(0) - 1)
    def _fin():
        sb = jnp.dot(accs[...], bmat,
                     preferred_element_type=jnp.float32)
        out_ref[...] = accp[...] / (sb + 1e-8)


def kernel(x, batch, query, W_k, b_k, W_v, b_v):
    scale = HD ** -0.5
    wkt = W_k.T
    a4 = scale * (wkt.reshape(DIM, H, HD) * query[None, :, :]).sum(-1)
    a8 = jnp.pad(a4, ((0, 0), (0, 4)))
    c4 = scale * (b_k.reshape(H, HD) * query).sum(-1)
    c8 = jnp.pad(c4, (0, 4)).reshape(1, 8)
    wvt = W_v.T
    bv = b_v.reshape(1, DIM)
    nblk = N_ROWS // BLK
    batchf = batch.astype(jnp.float32).reshape(nblk, 1, BLK)

    grid = (nblk,)
    out = pl.pallas_call(
        _tc_body,
        grid=grid,
        in_specs=[
            pl.BlockSpec((1, 1, BLK), lambda i: (i, 0, 0)),
            pl.BlockSpec((BLK, DIM), lambda i: (i, 0)),
            pl.BlockSpec((DIM, DIM), lambda i: (0, 0)),
            pl.BlockSpec((DIM, 8), lambda i: (0, 0)),
            pl.BlockSpec((1, 8), lambda i: (0, 0)),
            pl.BlockSpec((1, DIM), lambda i: (0, 0)),
        ],
        out_specs=pl.BlockSpec((BSZ, DIM), lambda i: (0, 0)),
        out_shape=jax.ShapeDtypeStruct((BSZ, DIM), jnp.float32),
        scratch_shapes=[
            pltpu.VMEM((BSZ, DIM), jnp.float32),
            pltpu.VMEM((BSZ, 8), jnp.float32),
        ],
    )(batchf, x, wvt, a8, c8, bv)
    return out
